# baseline (speedup 1.0000x reference)
import functools

import jax
import jax.numpy as jnp
from jax import lax
from jax.experimental import pallas as pl
from jax.experimental.pallas import tpu as pltpu
from jax.experimental.pallas import tpu_sc as plsc

CHUNK = 128
NBUF = 4
DP = 128


@functools.cache
def _build(B, V):
    info = plsc.get_sparse_core_info()
    nw = info.num_cores * info.num_subcores
    assert B % (nw * CHUNK * NBUF) == 0
    b_per_w = B // nw
    n_groups = b_per_w // (CHUNK * NBUF)
    mesh = plsc.VectorSubcoreMesh(core_axis_name="c", subcore_axis_name="s")

    @functools.partial(
        pl.kernel,
        out_type=jax.ShapeDtypeStruct((B, DP), jnp.float32),
        mesh=mesh,
        scratch_types=[
            pltpu.VMEM((b_per_w,), jnp.int32),
            pltpu.VMEM((NBUF, CHUNK, DP), jnp.float32),
            pltpu.SemaphoreType.DMA((NBUF,)),
            pltpu.SemaphoreType.DMA((NBUF,)),
        ],
    )
    def k(x_hbm, tab_hbm, out_hbm, idx_v, rows_v, gsem, wsem):
        wid = lax.axis_index("s") * info.num_cores + lax.axis_index("c")
        base = wid * b_per_w
        pltpu.sync_copy(x_hbm.at[pl.ds(base, b_per_w)], idx_v)

        def gather(j, b):
            return pltpu.make_async_copy(
                tab_hbm.at[idx_v.at[pl.ds(j * CHUNK, CHUNK)]],
                rows_v.at[b],
                gsem.at[b],
            )

        def write(j, b):
            return pltpu.make_async_copy(
                rows_v.at[b],
                out_hbm.at[pl.ds(base + j * CHUNK, CHUNK)],
                wsem.at[b],
            )

        for b in range(NBUF):
            gather(b, b).start()

        @pl.loop(0, n_groups)
        def _(g):
            j0 = g * NBUF
            for b in range(NBUF):
                gather(j0 + b, b).wait()
                write(j0 + b, b).start()
            for b in range(NBUF):
                write(j0 + b, b).wait()

                @pl.when(g + 1 < n_groups)
                def _():
                    gather(j0 + NBUF + b, b).start()

    return k


def kernel(x, W_embed):
    batch, hist = x.shape
    V, D = W_embed.shape
    flat = x.reshape(batch * hist).astype(jnp.int32)
    Wp = jnp.concatenate([W_embed, jnp.zeros((V, DP - D), jnp.float32)], axis=1)
    out_pad = _build(batch * hist, V)(flat, Wp)
    return out_pad.reshape(batch, hist, DP)[:, :, :D]


# speedup vs baseline: 1.0026x; 1.0026x over previous
import functools

import jax
import jax.numpy as jnp
from jax import lax
from jax.experimental import pallas as pl
from jax.experimental.pallas import tpu as pltpu
from jax.experimental.pallas import tpu_sc as plsc

CHUNK = 128
NBUF = 5
DP = 128


@functools.cache
def _build(B, V):
    info = plsc.get_sparse_core_info()
    nw = info.num_cores * info.num_subcores
    assert B % (nw * CHUNK * NBUF) == 0
    b_per_w = B // nw
    n_groups = b_per_w // (CHUNK * NBUF)
    mesh = plsc.VectorSubcoreMesh(core_axis_name="c", subcore_axis_name="s")

    @functools.partial(
        pl.kernel,
        out_type=jax.ShapeDtypeStruct((B, DP), jnp.float32),
        mesh=mesh,
        scratch_types=[
            pltpu.VMEM((b_per_w,), jnp.int32),
            pltpu.VMEM((NBUF, CHUNK, DP), jnp.float32),
            pltpu.SemaphoreType.DMA((NBUF,)),
            pltpu.SemaphoreType.DMA((NBUF,)),
        ],
    )
    def k(x_hbm, tab_hbm, out_hbm, idx_v, rows_v, gsem, wsem):
        wid = lax.axis_index("s") * info.num_cores + lax.axis_index("c")
        base = wid * b_per_w
        pltpu.sync_copy(x_hbm.at[pl.ds(base, b_per_w)], idx_v)

        def gather(j, b):
            return pltpu.make_async_copy(
                tab_hbm.at[idx_v.at[pl.ds(j * CHUNK, CHUNK)]],
                rows_v.at[b],
                gsem.at[b],
            )

        def write(j, b):
            return pltpu.make_async_copy(
                rows_v.at[b],
                out_hbm.at[pl.ds(base + j * CHUNK, CHUNK)],
                wsem.at[b],
            )

        for b in range(NBUF):
            gather(b, b).start()

        @pl.loop(0, n_groups)
        def _(g):
            j0 = g * NBUF
            for b in range(NBUF):
                gather(j0 + b, b).wait()
                write(j0 + b, b).start()
            for b in range(NBUF):
                write(j0 + b, b).wait()

                @pl.when(g + 1 < n_groups)
                def _():
                    gather(j0 + NBUF + b, b).start()

    return k


def kernel(x, W_embed):
    batch, hist = x.shape
    V, D = W_embed.shape
    flat = x.reshape(batch * hist).astype(jnp.int32)
    Wp = jnp.pad(W_embed, ((0, 0), (0, DP - D)))
    out_pad = _build(batch * hist, V)(flat, Wp)
    return out_pad.reshape(batch, hist, DP)[:, :, :D]
